# Initial kernel scaffold; baseline (speedup 1.0000x reference)
import jax
import jax.numpy as jnp
from jax.experimental import pallas as pl


def kernel(hidden_states, W):
    raise NotImplementedError("write your pallas kernel here")


# speedup vs baseline: 2.1774x; 2.1774x over previous
import jax
import jax.numpy as jnp
from jax.experimental import pallas as pl
from jax.experimental.pallas import tpu as pltpu

_B, _S, _H, _E, _K = 4, 4096, 4096, 64, 8
_AUX_W = 0.01
_N = _B * _S
_BLK = 512
_GRID = _N // _BLK


def _fused_body(hs_ref, w_ref, disp_ref, comb_ref, aux_ref, esum_ref):
    i = pl.program_id(0)
    logits = jax.lax.dot_general(
        hs_ref[...], w_ref[...], (((1,), (1,)), ((), ())),
        preferred_element_type=jnp.float32)

    lane = jax.lax.broadcasted_iota(jnp.int32, (_BLK, _E), 1)
    work = logits
    vals, idxs = [], []
    for _ in range(_K):
        m = jnp.max(work, axis=1, keepdims=True)
        amax = jnp.min(jnp.where(work == m, lane, _E), axis=1, keepdims=True)
        vals.append(m)
        idxs.append(amax)
        work = jnp.where(lane == amax, -jnp.inf, work)
    sel_vals = jnp.concatenate(vals, axis=1)
    sel_idx = jnp.concatenate(idxs, axis=1)

    ex = jnp.exp(sel_vals - sel_vals[:, 0:1])
    wn = ex / jnp.sum(ex, axis=1, keepdims=True)
    comb_ref[...] = wn

    e_iota = jax.lax.broadcasted_iota(jnp.int32, (_BLK, _K, _E), 2)
    mask3 = (sel_idx[:, :, None] == e_iota).astype(jnp.float32)
    disp_ref[...] = mask3

    es2 = jnp.sum(mask3 * wn[:, :, None], axis=1)
    es = jnp.sum(es2, axis=0, keepdims=True)

    @pl.when(i == 0)
    def _():
        esum_ref[...] = jnp.zeros_like(esum_ref)

    esum_ref[...] += es

    @pl.when(i == _GRID - 1)
    def _():
        s = esum_ref[0, :]
        aux_ref[0, 0] = jnp.sum(s * s) * (_AUX_W / _N)


def kernel(hidden_states, W):
    hs = hidden_states.reshape(_N, _H)
    disp, comb, aux = pl.pallas_call(
        _fused_body,
        grid=(_GRID,),
        in_specs=[
            pl.BlockSpec((_BLK, _H), lambda i: (i, 0)),
            pl.BlockSpec((_E, _H), lambda i: (0, 0)),
        ],
        out_specs=[
            pl.BlockSpec((_BLK, _K, _E), lambda i: (i, 0, 0)),
            pl.BlockSpec((_BLK, _K), lambda i: (i, 0)),
            pl.BlockSpec(memory_space=pltpu.SMEM),
        ],
        out_shape=[
            jax.ShapeDtypeStruct((_N, _K, _E), jnp.float32),
            jax.ShapeDtypeStruct((_N, _K), jnp.float32),
            jax.ShapeDtypeStruct((1, 1), jnp.float32),
        ],
        scratch_shapes=[pltpu.VMEM((1, _E), jnp.float32)],
    )(hs, W)
    combine_weights = comb.reshape(_B, _S, _K, 1)
    return disp, combine_weights, aux[0, 0]


# speedup vs baseline: 2.3812x; 1.0936x over previous
import functools

import jax
import jax.numpy as jnp
from jax import lax
from jax.experimental import pallas as pl
from jax.experimental.pallas import tpu as pltpu
from jax.experimental.pallas import tpu_sc as plsc

_B, _S, _H, _E, _K = 4, 4096, 4096, 64, 8
_AUX_W = 0.01
_N = _B * _S

_MM_BLK = 512
_MM_GRID = _N // _MM_BLK


def _mm_body(hs_ref, w_ref, out_ref):
    out_ref[...] = jax.lax.dot_general(
        hs_ref[...], w_ref[...], (((1,), (1,)), ((), ())),
        preferred_element_type=jnp.float32)


_NC, _NS, _L = 2, 16, 16
_NW = _NC * _NS
_TPW = _N // _NW
_CH = 64
_NCH = _TPW // _CH

_sc_mesh = plsc.VectorSubcoreMesh(
    core_axis_name="c", subcore_axis_name="s", num_cores=_NC, num_subcores=_NS)


@functools.partial(
    pl.kernel,
    out_type=[
        jax.ShapeDtypeStruct((_N * 512,), jnp.float32),
        jax.ShapeDtypeStruct((_N * _K,), jnp.float32),
        jax.ShapeDtypeStruct((_NW * _E,), jnp.float32),
    ],
    mesh=_sc_mesh,
    compiler_params=pltpu.CompilerParams(needs_layout_passes=False),
    scratch_types=[
        pltpu.VMEM((_CH * _E,), jnp.float32),
        pltpu.VMEM((_CH * _E,), jnp.float32),
        pltpu.VMEM((_CH * 512,), jnp.float32),
        pltpu.VMEM((_CH * 512,), jnp.float32),
        pltpu.VMEM((_CH * _K,), jnp.float32),
        pltpu.VMEM((_CH * _K,), jnp.float32),
        pltpu.VMEM((_CH * 16,), jnp.int32),
        pltpu.VMEM((_CH * 16,), jnp.int32),
        pltpu.VMEM((_E,), jnp.float32),
        pltpu.SemaphoreType.DMA,
        pltpu.SemaphoreType.DMA,
        pltpu.SemaphoreType.DMA,
        pltpu.SemaphoreType.DMA,
        pltpu.SemaphoreType.DMA,
        pltpu.SemaphoreType.DMA,
    ],
)
def _sc_router(log_hbm, disp_hbm, comb_hbm, esum_hbm,
               log_v0, log_v1, disp_v0, disp_v1, comb_v0, comb_v1,
               idx_v0, idx_v1, acc_v,
               lin0, lin1, dout0, dout1, cout0, cout1):
    cid = lax.axis_index("c")
    sid = lax.axis_index("s")
    wid = sid * _NC + cid
    base = wid * _TPW

    lane = lax.iota(jnp.int32, 16)
    lane_lt8 = lane < _K
    zero16 = jnp.zeros((16,), jnp.float32)
    one16 = jnp.ones((16,), jnp.float32)
    vals_base = [lane, lane + 16, lane + 32, lane + 48]

    log_v = (log_v0, log_v1)
    disp_v = (disp_v0, disp_v1)
    comb_v = (comb_v0, comb_v1)
    idx_v = (idx_v0, idx_v1)
    lin = (lin0, lin1)
    dout = (dout0, dout1)
    cout = (cout0, cout1)

    def _memset(ref, words):
        def mbody(i, carry):
            b = i * 128
            for j in range(8):
                ref[pl.ds(b + j * 16, 16)] = zero16
            return carry
        lax.fori_loop(0, words // 128, mbody, 0)

    _memset(disp_v0, _CH * 512)
    _memset(disp_v1, _CH * 512)
    for j in range(_E // 16):
        acc_v[pl.ds(j * 16, 16)] = zero16

    def _log_slice(c):
        return log_hbm.at[pl.ds(base * _E + c * (_CH * _E), _CH * _E)]

    def _disp_slice(c):
        return disp_hbm.at[pl.ds((base + c * _CH) * 512, _CH * 512)]

    def _comb_slice(c):
        return comb_hbm.at[pl.ds((base + c * _CH) * _K, _CH * _K)]

    pltpu.async_copy(_log_slice(0), log_v0, lin0)
    pltpu.async_copy(_log_slice(1), log_v1, lin1)

    def _merge(a, b):
        mk = jnp.where(lane_lt8, a[0], lax.rev(b[0], (0,)))
        mv = jnp.where(lane_lt8, a[1], lax.rev(b[1], (0,)))
        return plsc.sort_key_val(mk, mv, descending=True)

    def _chunk(c, s):
        lv, dv, cv, iv = log_v[s], disp_v[s], comb_v[s], idx_v[s]
        pltpu.make_async_copy(_log_slice(c), lv, lin[s]).wait()

        def tok(t, carry):
            off = t * _E
            pairs = [
                plsc.sort_key_val(lv[pl.ds(off + 16 * j, 16)], vals_base[j],
                                  descending=True)
                for j in range(4)
            ]
            fk, fv = _merge(_merge(pairs[0], pairs[1]),
                            _merge(pairs[2], pairs[3]))
            kmax = jnp.max(fk)
            ex = jnp.where(lane_lt8, jnp.exp(fk - kmax), 0.0)
            wn = ex / jnp.sum(ex)
            sidx = t * 512 + lane * _E + fv
            plsc.store_scatter(dv, [sidx], one16, mask=lane_lt8)
            iv[pl.ds(t * 16, 16)] = sidx
            plsc.store_scatter(cv, [t * _K + lane], wn, mask=lane_lt8)
            plsc.addupdate_scatter(acc_v, [fv], wn, mask=lane_lt8)
            return carry

        lax.fori_loop(0, _CH, tok, 0)
        pltpu.async_copy(dv, _disp_slice(c), dout[s])
        pltpu.async_copy(cv, _comb_slice(c), cout[s])

    def _cleanup(c, s):
        dv, cv, iv = disp_v[s], comb_v[s], idx_v[s]
        pltpu.make_async_copy(dv, _disp_slice(c), dout[s]).wait()
        pltpu.make_async_copy(cv, _comb_slice(c), cout[s]).wait()

        def clb(t, carry):
            sidx = iv[pl.ds(t * 16, 16)]
            plsc.store_scatter(dv, [sidx], zero16, mask=lane_lt8)
            return carry

        lax.fori_loop(0, _CH, clb, 0)

    def mloop(m, carry):
        for s in range(2):
            c = m * 2 + s

            @pl.when(m > 0)
            def _():
                _cleanup(c - 2, s)

            _chunk(c, s)

            @pl.when(m < _NCH // 2 - 1)
            def _():
                pltpu.async_copy(_log_slice(c + 2), log_v[s], lin[s])

        return carry

    lax.fori_loop(0, _NCH // 2, mloop, 0)

    for s in range(2):
        c = _NCH - 2 + s
        pltpu.make_async_copy(disp_v[s], _disp_slice(c), dout[s]).wait()
        pltpu.make_async_copy(comb_v[s], _comb_slice(c), cout[s]).wait()

    pltpu.sync_copy(acc_v, esum_hbm.at[pl.ds(wid * _E, _E)])


def _aux_body(es_ref, aux_ref):
    s = jnp.sum(es_ref[...], axis=0)
    aux_ref[0, 0] = jnp.sum(s * s) * (_AUX_W / _N)


def kernel(hidden_states, W):
    hs = hidden_states.reshape(_N, _H)
    logits = pl.pallas_call(
        _mm_body,
        grid=(_MM_GRID,),
        in_specs=[
            pl.BlockSpec((_MM_BLK, _H), lambda i: (i, 0)),
            pl.BlockSpec((_E, _H), lambda i: (0, 0)),
        ],
        out_specs=pl.BlockSpec((_MM_BLK, _E), lambda i: (i, 0)),
        out_shape=jax.ShapeDtypeStruct((_N, _E), jnp.float32),
    )(hs, W)

    disp, comb, esum = _sc_router(logits.reshape(_N * _E))

    aux = pl.pallas_call(
        _aux_body,
        in_specs=[pl.BlockSpec((_NW, _E), lambda: (0, 0))],
        out_specs=pl.BlockSpec(memory_space=pltpu.SMEM),
        out_shape=jax.ShapeDtypeStruct((1, 1), jnp.float32),
    )(esum.reshape(_NW, _E))

    dispatch_mask = disp.reshape(_N, _K, _E)
    combine_weights = comb.reshape(_B, _S, _K, 1)
    return dispatch_mask, combine_weights, aux[0, 0]
